# Initial kernel scaffold; baseline (speedup 1.0000x reference)
import jax
import jax.numpy as jnp
from jax.experimental import pallas as pl


def kernel(x_cell, x_gene, edge_index_ce, edge_index_ec, Wl1_cg, bl1_cg, Wr1_cg, Wl1_gc, bl1_gc, Wr1_gc, Wl2_cg, bl2_cg, Wr2_cg, Wl2_gc, bl2_gc, Wr2_gc, W_lin, b_lin):
    raise NotImplementedError("write your pallas kernel here")


# speedup vs baseline: 1.8399x; 1.8399x over previous
import functools
import jax
import jax.numpy as jnp
from jax import lax
from jax.experimental import pallas as pl
from jax.experimental.pallas import tpu as pltpu
from jax.experimental.pallas import tpu_sc as plsc

N = 10000
D = 128
H = 256
OUT = 64
E = 320000
EPAD = 327680
NPAD = 10112
EB = 128

_mesh = plsc.VectorSubcoreMesh(core_axis_name="c", subcore_axis_name="s")


def _wb(s, sh, out, c):
    @pl.when(s < 15)
    def _():
        pltpu.sync_copy(sh.at[pl.ds(s * 632, 632)],
                        out.at[c].at[pl.ds(s * 632, 632)])

    @pl.when(s == 15)
    def _():
        pltpu.sync_copy(sh.at[pl.ds(9480, 520)],
                        out.at[c].at[pl.ds(9480, 520)])


def _sc_seg1(src_hbm, dst_hbm, x_hbm, zf_hbm, ones_hbm,
             agg_out, cnt_out,
             shf, src_v, dst_v, rows_v, ones_v, sem):
    c = lax.axis_index("c")
    s = lax.axis_index("s")
    pltpu.sync_copy(zf_hbm.at[pl.ds(s * 632, 632)], shf.at[pl.ds(s * 632, 632)])
    pltpu.sync_copy(ones_hbm, ones_v)
    plsc.subcore_barrier()

    base = (c * 16 + s) * 10240

    def step(i, carry):
        off = base + i * EB
        pltpu.sync_copy(src_hbm.at[pl.ds(off, EB)], src_v)
        pltpu.sync_copy(dst_hbm.at[pl.ds(off, EB)], dst_v)
        pltpu.async_copy(x_hbm.at[src_v], rows_v, sem).wait()
        pltpu.sync_copy(rows_v, shf.at[dst_v], add=True)
        return carry

    lax.fori_loop(0, 10240 // EB, step, 0)
    plsc.subcore_barrier()
    _wb(s, shf, agg_out, c)
    plsc.subcore_barrier()

    pltpu.sync_copy(zf_hbm.at[pl.ds(s * 632, 632)], shf.at[pl.ds(s * 632, 632)])
    plsc.subcore_barrier()

    def stepc(i, carry):
        off = base + i * EB
        pltpu.sync_copy(dst_hbm.at[pl.ds(off, EB)], dst_v)
        pltpu.sync_copy(ones_v, shf.at[dst_v], add=True)
        return carry

    lax.fori_loop(0, 10240 // EB, stepc, 0)
    plsc.subcore_barrier()
    _wb(s, shf, cnt_out, c)


def _seg1(src, dst, x, zf, ones):
    return pl.kernel(
        _sc_seg1,
        out_type=(
            jax.ShapeDtypeStruct((2, N, D), jnp.float32),
            jax.ShapeDtypeStruct((2, N, D), jnp.float32),
        ),
        mesh=_mesh,
        scratch_types=(
            pltpu.VMEM_SHARED((NPAD, D), jnp.float32),
            pltpu.VMEM((EB,), jnp.int32),
            pltpu.VMEM((EB,), jnp.int32),
            pltpu.VMEM((EB, D), jnp.float32),
            pltpu.VMEM((EB, D), jnp.float32),
            pltpu.SemaphoreType.DMA,
        ),
    )(src, dst, x, zf, ones)


def _sc_seg2(src2_hbm, dst_hbm, h_hbm, zf_hbm,
             agg_out,
             shf, src_v, dst_v, rows_v, sem):
    c = lax.axis_index("c")
    s = lax.axis_index("s")
    pltpu.sync_copy(zf_hbm.at[pl.ds(s * 632, 632)], shf.at[pl.ds(s * 632, 632)])
    plsc.subcore_barrier()

    base = s * 20480

    def step(i, carry):
        off = base + i * EB
        soff = pl.multiple_of(c * EPAD + off, EB)
        pltpu.sync_copy(src2_hbm.at[pl.ds(soff, EB)], src_v)
        pltpu.sync_copy(dst_hbm.at[pl.ds(off, EB)], dst_v)
        pltpu.async_copy(h_hbm.at[src_v], rows_v, sem).wait()
        pltpu.sync_copy(rows_v, shf.at[dst_v], add=True)
        return carry

    lax.fori_loop(0, 20480 // EB, step, 0)
    plsc.subcore_barrier()

    @pl.when(s < 15)
    def _():
        pltpu.sync_copy(shf.at[pl.ds(s * 632, 632)],
                        agg_out.at[c].at[pl.ds(s * 632, 632)])

    @pl.when(s == 15)
    def _():
        pltpu.sync_copy(shf.at[pl.ds(9480, 520)],
                        agg_out.at[c].at[pl.ds(9480, 520)])


def _seg2(src2, dst, h_both, zf):
    return pl.kernel(
        _sc_seg2,
        out_type=jax.ShapeDtypeStruct((2, N, D), jnp.float32),
        mesh=_mesh,
        scratch_types=(
            pltpu.VMEM_SHARED((NPAD, D), jnp.float32),
            pltpu.VMEM((EB,), jnp.int32),
            pltpu.VMEM((EB,), jnp.int32),
            pltpu.VMEM((EB, D), jnp.float32),
            pltpu.SemaphoreType.DMA,
        ),
    )(src2, dst, h_both, zf)


BN = 1000


def _tc1_body(p_ref, cp_ref, x_ref, wl_ref, wr_ref, b_ref, out_ref):
    cnt = cp_ref[0, :, 0:1] + cp_ref[1, :, 0:1]
    inv = 1.0 / jnp.maximum(cnt, 1.0)
    mean = (p_ref[0] + p_ref[1]) * inv
    r = (jnp.dot(mean, wl_ref[...], preferred_element_type=jnp.float32)
         + jnp.dot(x_ref[...], wr_ref[...], preferred_element_type=jnp.float32)
         + b_ref[...])
    h = jnp.maximum(r, 0.0)
    out_ref[0] = h[:, :D]
    out_ref[1] = h[:, D:]


def _tc1(p, cp, x, wlT, wrT, b):
    return pl.pallas_call(
        _tc1_body,
        grid=(N // BN,),
        in_specs=[
            pl.BlockSpec((2, BN, D), lambda i: (0, i, 0)),
            pl.BlockSpec((2, BN, D), lambda i: (0, i, 0)),
            pl.BlockSpec((BN, D), lambda i: (i, 0)),
            pl.BlockSpec((D, H), lambda i: (0, 0)),
            pl.BlockSpec((D, H), lambda i: (0, 0)),
            pl.BlockSpec((1, H), lambda i: (0, 0)),
        ],
        out_specs=pl.BlockSpec((2, BN, D), lambda i: (0, i, 0)),
        out_shape=jax.ShapeDtypeStruct((2, N, D), jnp.float32),
    )(p, cp, x, wlT, wrT, b)


def _tc2_body(a_ref, cp_ref, h_ref, wl_ref, wr_ref, b_ref, out_ref):
    cnt = cp_ref[0, :, 0:1] + cp_ref[1, :, 0:1]
    inv = 1.0 / jnp.maximum(cnt, 1.0)
    r = (jnp.dot(a_ref[0] * inv, wl_ref[:D], preferred_element_type=jnp.float32)
         + jnp.dot(a_ref[1] * inv, wl_ref[D:], preferred_element_type=jnp.float32)
         + jnp.dot(h_ref[0], wr_ref[:D], preferred_element_type=jnp.float32)
         + jnp.dot(h_ref[1], wr_ref[D:], preferred_element_type=jnp.float32)
         + b_ref[...])
    out_ref[...] = jnp.maximum(r, 0.0)


def _tc2(a, cp, h, wlT, wrT, b):
    return pl.pallas_call(
        _tc2_body,
        grid=(N // BN,),
        in_specs=[
            pl.BlockSpec((2, BN, D), lambda i: (0, i, 0)),
            pl.BlockSpec((2, BN, D), lambda i: (0, i, 0)),
            pl.BlockSpec((2, BN, D), lambda i: (0, i, 0)),
            pl.BlockSpec((H, H), lambda i: (0, 0)),
            pl.BlockSpec((H, H), lambda i: (0, 0)),
            pl.BlockSpec((1, H), lambda i: (0, 0)),
        ],
        out_specs=pl.BlockSpec((BN, H), lambda i: (i, 0)),
        out_shape=jax.ShapeDtypeStruct((N, H), jnp.float32),
    )(a, cp, h, wlT, wrT, b)


def _tc2fin_body(a_ref, cp_ref, h_ref, wl_ref, wr_ref, b_ref, wlin_ref,
                 blin_ref, out_ref, log_ref):
    cnt = cp_ref[0, :, 0:1] + cp_ref[1, :, 0:1]
    inv = 1.0 / jnp.maximum(cnt, 1.0)
    r = (jnp.dot(a_ref[0] * inv, wl_ref[:D], preferred_element_type=jnp.float32)
         + jnp.dot(a_ref[1] * inv, wl_ref[D:], preferred_element_type=jnp.float32)
         + jnp.dot(h_ref[0], wr_ref[:D], preferred_element_type=jnp.float32)
         + jnp.dot(h_ref[1], wr_ref[D:], preferred_element_type=jnp.float32)
         + b_ref[...])
    c2 = jnp.maximum(r, 0.0)
    out_ref[...] = c2
    log_ref[...] = (jnp.dot(c2, wlin_ref[...], preferred_element_type=jnp.float32)
                    + blin_ref[...])


def _tc2fin(a, cp, h, wlT, wrT, b, wlinT, blin):
    return pl.pallas_call(
        _tc2fin_body,
        grid=(N // BN,),
        in_specs=[
            pl.BlockSpec((2, BN, D), lambda i: (0, i, 0)),
            pl.BlockSpec((2, BN, D), lambda i: (0, i, 0)),
            pl.BlockSpec((2, BN, D), lambda i: (0, i, 0)),
            pl.BlockSpec((H, H), lambda i: (0, 0)),
            pl.BlockSpec((H, H), lambda i: (0, 0)),
            pl.BlockSpec((1, H), lambda i: (0, 0)),
            pl.BlockSpec((H, OUT), lambda i: (0, 0)),
            pl.BlockSpec((1, OUT), lambda i: (0, 0)),
        ],
        out_specs=(
            pl.BlockSpec((BN, H), lambda i: (i, 0)),
            pl.BlockSpec((BN, OUT), lambda i: (i, 0)),
        ),
        out_shape=(
            jax.ShapeDtypeStruct((N, H), jnp.float32),
            jax.ShapeDtypeStruct((N, OUT), jnp.float32),
        ),
    )(a, cp, h, wlT, wrT, b, wlinT, blin)


def kernel(x_cell, x_gene, edge_index_ce, edge_index_ec,
           Wl1_cg, bl1_cg, Wr1_cg, Wl1_gc, bl1_gc, Wr1_gc,
           Wl2_cg, bl2_cg, Wr2_cg, Wl2_gc, bl2_gc, Wr2_gc,
           W_lin, b_lin):
    npad = EPAD - E
    pad_src = jnp.zeros((npad,), jnp.int32)
    pad_dst = jnp.full((npad,), N, jnp.int32)

    src_ce = jnp.concatenate([edge_index_ce[0].astype(jnp.int32), pad_src])
    dst_ce = jnp.concatenate([edge_index_ce[1].astype(jnp.int32), pad_dst])
    src_ec = jnp.concatenate([edge_index_ec[0].astype(jnp.int32), pad_src])
    dst_ec = jnp.concatenate([edge_index_ec[1].astype(jnp.int32), pad_dst])

    zf = jnp.zeros((NPAD, D), jnp.float32)
    ones = jnp.ones((EB, D), jnp.float32)

    agg1g, cnt_ce = _seg1(src_ce, dst_ce, x_cell, zf, ones)
    agg1c, cnt_ec = _seg1(src_ec, dst_ec, x_gene, zf, ones)

    b1cg = bl1_cg.reshape(1, H)
    b1gc = bl1_gc.reshape(1, H)
    g1 = _tc1(agg1g, cnt_ce, x_gene, Wl1_cg.T, Wr1_cg.T, b1cg)
    c1 = _tc1(agg1c, cnt_ec, x_cell, Wl1_gc.T, Wr1_gc.T, b1gc)

    src2_ce = jnp.concatenate([src_ce, src_ce + N])
    src2_ec = jnp.concatenate([src_ec, src_ec + N])
    a2g = _seg2(src2_ce, dst_ce, c1.reshape(2 * N, D), zf)
    a2c = _seg2(src2_ec, dst_ec, g1.reshape(2 * N, D), zf)

    g2 = _tc2(a2g, cnt_ce, g1, Wl2_cg.T, Wr2_cg.T, bl2_cg.reshape(1, H))
    c2, logits = _tc2fin(a2c, cnt_ec, c1, Wl2_gc.T, Wr2_gc.T,
                         bl2_gc.reshape(1, H), W_lin.T, b_lin.reshape(1, OUT))
    return (logits, c2, g2)


# speedup vs baseline: 2.6239x; 1.4261x over previous
import jax
import jax.numpy as jnp
from jax import lax
from jax.experimental import pallas as pl
from jax.experimental.pallas import tpu as pltpu
from jax.experimental.pallas import tpu_sc as plsc

N = 10000
D = 128
H = 256
OUT = 64
E = 320000
EPAD = 327680
NPAD = 10112
EB = 128
K = 2
NB = 160
NG = NB // K

_mesh = plsc.VectorSubcoreMesh(core_axis_name="c", subcore_axis_name="s")


def _wb(s, sh, out, c):
    @pl.when(s < 15)
    def _():
        pltpu.sync_copy(sh.at[pl.ds(s * 632, 632)],
                        out.at[c].at[pl.ds(s * 632, 632)])

    @pl.when(s == 15)
    def _():
        pltpu.sync_copy(sh.at[pl.ds(9480, 520)],
                        out.at[c].at[pl.ds(9480, 520)])


def _fire_idx(base, g, half, srcf, dstf, srcv, dstv, isem):
    for b in range(K):
        off = base + (g * K + b) * EB
        pltpu.async_copy(srcf.at[pl.ds(off, EB)], srcv.at[half * K + b], isem)
        pltpu.async_copy(dstf.at[pl.ds(off, EB)], dstv.at[half * K + b], isem)


def _drain(src_dummy, dst_ref, sem):
    pltpu.make_async_copy(src_dummy, dst_ref, sem).wait()


def _gather_scatter_loop(base, table, srcf, dstf, shf,
                         srcv, dstv, rows, isem, gsem, ssem):
    _fire_idx(base, 0, 0, srcf, dstf, srcv, dstv, isem)

    def one_group(j, half):
        for b in range(K):
            _drain(srcf.at[pl.ds(base, EB)], srcv.at[half * K + b], isem)
            _drain(dstf.at[pl.ds(base, EB)], dstv.at[half * K + b], isem)
        _fire_idx(base, j + 1, 1 - half, srcf, dstf, srcv, dstv, isem)
        for b in range(K):
            pltpu.async_copy(table.at[srcv.at[half * K + b]], rows.at[b], gsem)
        for b in range(K):
            _drain(table.at[pl.ds(0, EB)], rows.at[b], gsem)
        for b in range(K):
            pltpu.async_copy(rows.at[b], shf.at[dstv.at[half * K + b]],
                             ssem, add=True)
        for b in range(K):
            _drain(table.at[pl.ds(0, EB)], rows.at[b], ssem)

    def pair(t, carry):
        one_group(2 * t, 0)
        one_group(2 * t + 1, 1)
        return carry

    lax.fori_loop(0, NG // 2, pair, 0)
    for b in range(K):
        _drain(srcf.at[pl.ds(base, EB)], srcv.at[b], isem)
        _drain(dstf.at[pl.ds(base, EB)], dstv.at[b], isem)


def _fire_dst(base, g, half, dstf, dstv, isem):
    for b in range(K):
        off = base + (g * K + b) * EB
        pltpu.async_copy(dstf.at[pl.ds(off, EB)], dstv.at[half * K + b], isem)


def _count_loop(base, dstf, shf, zf_hbm, ones_v, dstv, isem, ssem):
    _fire_dst(base, 0, 0, dstf, dstv, isem)

    def one_group(j, half):
        for b in range(K):
            _drain(dstf.at[pl.ds(base, EB)], dstv.at[half * K + b], isem)
        _fire_dst(base, j + 1, 1 - half, dstf, dstv, isem)
        for b in range(K):
            pltpu.async_copy(ones_v, shf.at[dstv.at[half * K + b]],
                             ssem, add=True)
        for b in range(K):
            _drain(zf_hbm.at[pl.ds(0, EB)], ones_v, ssem)

    def pair(t, carry):
        one_group(2 * t, 0)
        one_group(2 * t + 1, 1)
        return carry

    lax.fori_loop(0, NG // 2, pair, 0)
    for b in range(K):
        _drain(dstf.at[pl.ds(base, EB)], dstv.at[b], isem)


def _sc_seg_cnt(srcf_hbm, dstf_hbm, table_hbm, zf_hbm, ones_hbm,
                agg_out, cnt_out,
                shf, srcv, dstv, rows, isem, gsem, ssem):
    c = lax.axis_index("c")
    s = lax.axis_index("s")
    pltpu.sync_copy(zf_hbm.at[pl.ds(s * 632, 632)], shf.at[pl.ds(s * 632, 632)])
    plsc.subcore_barrier()
    base = c * EPAD + s * 20480
    _gather_scatter_loop(base, table_hbm, srcf_hbm, dstf_hbm, shf,
                         srcv, dstv, rows, isem, gsem, ssem)
    plsc.subcore_barrier()
    _wb(s, shf, agg_out, c)
    plsc.subcore_barrier()
    pltpu.sync_copy(zf_hbm.at[pl.ds(s * 632, 632)], shf.at[pl.ds(s * 632, 632)])
    pltpu.sync_copy(ones_hbm, rows.at[0])
    plsc.subcore_barrier()
    _count_loop(base, dstf_hbm, shf, zf_hbm, rows.at[0], dstv, isem, ssem)
    plsc.subcore_barrier()
    _wb(s, shf, cnt_out, c)


def _sc_seg(srcf_hbm, dstf_hbm, table_hbm, zf_hbm,
            agg_out,
            shf, srcv, dstv, rows, isem, gsem, ssem):
    c = lax.axis_index("c")
    s = lax.axis_index("s")
    pltpu.sync_copy(zf_hbm.at[pl.ds(s * 632, 632)], shf.at[pl.ds(s * 632, 632)])
    plsc.subcore_barrier()
    base = c * EPAD + s * 20480
    _gather_scatter_loop(base, table_hbm, srcf_hbm, dstf_hbm, shf,
                         srcv, dstv, rows, isem, gsem, ssem)
    plsc.subcore_barrier()
    _wb(s, shf, agg_out, c)


_SC_SCRATCH = (
    pltpu.VMEM_SHARED((NPAD, D), jnp.float32),
    pltpu.VMEM((2 * K, EB), jnp.int32),
    pltpu.VMEM((2 * K, EB), jnp.int32),
    pltpu.VMEM((K, EB, D), jnp.float32),
)
_SEMS = (pltpu.SemaphoreType.DMA,) * 3


def _seg_cnt(srcf, dstf, table, zf, ones):
    return pl.kernel(
        _sc_seg_cnt,
        out_type=(
            jax.ShapeDtypeStruct((2, N, D), jnp.float32),
            jax.ShapeDtypeStruct((2, N, D), jnp.float32),
        ),
        mesh=_mesh,
        scratch_types=_SC_SCRATCH + _SEMS,
    )(srcf, dstf, table, zf, ones)


def _seg(srcf, dstf, table, zf):
    return pl.kernel(
        _sc_seg,
        out_type=jax.ShapeDtypeStruct((2, N, D), jnp.float32),
        mesh=_mesh,
        scratch_types=_SC_SCRATCH + _SEMS,
    )(srcf, dstf, table, zf)


BN = 1000


def _tc1_body(p_ref, cp_ref, x_ref, wl_ref, wr_ref, b_ref, out_ref):
    cnt = cp_ref[0, :, 0:1]
    inv = 1.0 / jnp.maximum(cnt, 1.0)
    mean = p_ref[0] * inv
    r = (jnp.dot(mean, wl_ref[...], preferred_element_type=jnp.float32)
         + jnp.dot(x_ref[...], wr_ref[...], preferred_element_type=jnp.float32)
         + b_ref[...])
    h = jnp.maximum(r, 0.0)
    out_ref[0] = h[:, :D]
    out_ref[1] = h[:, D:]


def _tc1(rel, p, cp, x, wlT, wrT, b):
    return pl.pallas_call(
        _tc1_body,
        grid=(N // BN,),
        in_specs=[
            pl.BlockSpec((1, BN, D), lambda i, r=rel: (r, i, 0)),
            pl.BlockSpec((1, BN, D), lambda i, r=rel: (r, i, 0)),
            pl.BlockSpec((BN, D), lambda i: (i, 0)),
            pl.BlockSpec((D, H), lambda i: (0, 0)),
            pl.BlockSpec((D, H), lambda i: (0, 0)),
            pl.BlockSpec((1, H), lambda i: (0, 0)),
        ],
        out_specs=pl.BlockSpec((2, BN, D), lambda i: (0, i, 0)),
        out_shape=jax.ShapeDtypeStruct((2, N, D), jnp.float32),
    )(p, cp, x, wlT, wrT, b)


def _tc2_body(a_ref, cp_ref, h_ref, wl_ref, wr_ref, b_ref, out_ref):
    cnt = cp_ref[0, :, 0:1]
    inv = 1.0 / jnp.maximum(cnt, 1.0)
    r = (jnp.dot(a_ref[0] * inv, wl_ref[:D], preferred_element_type=jnp.float32)
         + jnp.dot(a_ref[1] * inv, wl_ref[D:], preferred_element_type=jnp.float32)
         + jnp.dot(h_ref[0], wr_ref[:D], preferred_element_type=jnp.float32)
         + jnp.dot(h_ref[1], wr_ref[D:], preferred_element_type=jnp.float32)
         + b_ref[...])
    out_ref[...] = jnp.maximum(r, 0.0)


def _tc2(rel, a, cp, h, wlT, wrT, b):
    return pl.pallas_call(
        _tc2_body,
        grid=(N // BN,),
        in_specs=[
            pl.BlockSpec((2, BN, D), lambda i: (0, i, 0)),
            pl.BlockSpec((1, BN, D), lambda i, r=rel: (r, i, 0)),
            pl.BlockSpec((2, BN, D), lambda i: (0, i, 0)),
            pl.BlockSpec((H, H), lambda i: (0, 0)),
            pl.BlockSpec((H, H), lambda i: (0, 0)),
            pl.BlockSpec((1, H), lambda i: (0, 0)),
        ],
        out_specs=pl.BlockSpec((BN, H), lambda i: (i, 0)),
        out_shape=jax.ShapeDtypeStruct((N, H), jnp.float32),
    )(a, cp, h, wlT, wrT, b)


def _tc2fin_body(a_ref, cp_ref, h_ref, wl_ref, wr_ref, b_ref, wlin_ref,
                 blin_ref, out_ref, log_ref):
    cnt = cp_ref[0, :, 0:1]
    inv = 1.0 / jnp.maximum(cnt, 1.0)
    r = (jnp.dot(a_ref[0] * inv, wl_ref[:D], preferred_element_type=jnp.float32)
         + jnp.dot(a_ref[1] * inv, wl_ref[D:], preferred_element_type=jnp.float32)
         + jnp.dot(h_ref[0], wr_ref[:D], preferred_element_type=jnp.float32)
         + jnp.dot(h_ref[1], wr_ref[D:], preferred_element_type=jnp.float32)
         + b_ref[...])
    c2 = jnp.maximum(r, 0.0)
    out_ref[...] = c2
    log_ref[...] = (jnp.dot(c2, wlin_ref[...], preferred_element_type=jnp.float32)
                    + blin_ref[...])


def _tc2fin(rel, a, cp, h, wlT, wrT, b, wlinT, blin):
    return pl.pallas_call(
        _tc2fin_body,
        grid=(N // BN,),
        in_specs=[
            pl.BlockSpec((2, BN, D), lambda i: (0, i, 0)),
            pl.BlockSpec((1, BN, D), lambda i, r=rel: (r, i, 0)),
            pl.BlockSpec((2, BN, D), lambda i: (0, i, 0)),
            pl.BlockSpec((H, H), lambda i: (0, 0)),
            pl.BlockSpec((H, H), lambda i: (0, 0)),
            pl.BlockSpec((1, H), lambda i: (0, 0)),
            pl.BlockSpec((H, OUT), lambda i: (0, 0)),
            pl.BlockSpec((1, OUT), lambda i: (0, 0)),
        ],
        out_specs=(
            pl.BlockSpec((BN, H), lambda i: (i, 0)),
            pl.BlockSpec((BN, OUT), lambda i: (i, 0)),
        ),
        out_shape=(
            jax.ShapeDtypeStruct((N, H), jnp.float32),
            jax.ShapeDtypeStruct((N, OUT), jnp.float32),
        ),
    )(a, cp, h, wlT, wrT, b, wlinT, blin)


def kernel(x_cell, x_gene, edge_index_ce, edge_index_ec,
           Wl1_cg, bl1_cg, Wr1_cg, Wl1_gc, bl1_gc, Wr1_gc,
           Wl2_cg, bl2_cg, Wr2_cg, Wl2_gc, bl2_gc, Wr2_gc,
           W_lin, b_lin):
    npad = EPAD - E
    tail = K * EB
    pad_src = jnp.zeros((npad,), jnp.int32)
    pad_dst = jnp.full((npad,), N, jnp.int32)
    zpad = jnp.zeros((tail,), jnp.int32)

    src_ce = jnp.concatenate([edge_index_ce[0].astype(jnp.int32), pad_src])
    dst_ce = jnp.concatenate([edge_index_ce[1].astype(jnp.int32), pad_dst])
    src_ec = jnp.concatenate([edge_index_ec[0].astype(jnp.int32), pad_src])
    dst_ec = jnp.concatenate([edge_index_ec[1].astype(jnp.int32), pad_dst])

    zf = jnp.zeros((NPAD, D), jnp.float32)
    ones = jnp.ones((EB, D), jnp.float32)

    x_both = jnp.concatenate([x_cell, x_gene], axis=0)
    src1 = jnp.concatenate([src_ce, src_ec + N, zpad])
    dst1 = jnp.concatenate([dst_ce, dst_ec, zpad])
    agg1, cnt1 = _seg_cnt(src1, dst1, x_both, zf, ones)

    g1 = _tc1(0, agg1, cnt1, x_gene, Wl1_cg.T, Wr1_cg.T, bl1_cg.reshape(1, H))
    c1 = _tc1(1, agg1, cnt1, x_cell, Wl1_gc.T, Wr1_gc.T, bl1_gc.reshape(1, H))

    src2_ce = jnp.concatenate([src_ce, src_ce + N, zpad])
    dst2_ce = jnp.concatenate([dst_ce, dst_ce, zpad])
    src2_ec = jnp.concatenate([src_ec, src_ec + N, zpad])
    dst2_ec = jnp.concatenate([dst_ec, dst_ec, zpad])
    a2g = _seg(src2_ce, dst2_ce, c1.reshape(2 * N, D), zf)
    a2c = _seg(src2_ec, dst2_ec, g1.reshape(2 * N, D), zf)

    g2 = _tc2(0, a2g, cnt1, g1, Wl2_cg.T, Wr2_cg.T, bl2_cg.reshape(1, H))
    c2, logits = _tc2fin(1, a2c, cnt1, c1, Wl2_gc.T, Wr2_gc.T,
                         bl2_gc.reshape(1, H), W_lin.T, b_lin.reshape(1, OUT))
    return (logits, c2, g2)


# speedup vs baseline: 2.7697x; 1.0555x over previous
import jax
import jax.numpy as jnp
from jax import lax
from jax.experimental import pallas as pl
from jax.experimental.pallas import tpu as pltpu
from jax.experimental.pallas import tpu_sc as plsc

N = 10000
D = 128
H = 256
OUT = 64
E = 320000
EPAD = 327680
NPAD = 10112
EB = 128
K = 2
NB = 160
NG = NB // K

_mesh = plsc.VectorSubcoreMesh(core_axis_name="c", subcore_axis_name="s")


def _wb(s, sh, out, c):
    @pl.when(s < 15)
    def _():
        pltpu.sync_copy(sh.at[pl.ds(s * 632, 632)],
                        out.at[c].at[pl.ds(s * 632, 632)])

    @pl.when(s == 15)
    def _():
        pltpu.sync_copy(sh.at[pl.ds(9480, 520)],
                        out.at[c].at[pl.ds(9480, 520)])


def _drain(src_dummy, dst_ref, sem):
    pltpu.make_async_copy(src_dummy, dst_ref, sem).wait()


def _gather_scatter_loop(base, table, srcf, dstf, shf,
                         srcv, dstv, rows, zf_hbm, isem, gsem, ssem):
    for q in range(3):
        off = base + q * EB
        pltpu.async_copy(srcf.at[pl.ds(off, EB)], srcv.at[q], isem)
        pltpu.async_copy(dstf.at[pl.ds(off, EB)], dstv.at[q], isem)
    pltpu.sync_copy(zf_hbm.at[pl.ds(0, EB)], rows.at[1])
    _drain(srcf.at[pl.ds(base, EB)], srcv.at[0], isem)
    _drain(dstf.at[pl.ds(base, EB)], dstv.at[0], isem)
    pltpu.async_copy(table.at[srcv.at[0]], rows.at[0], gsem)
    pltpu.async_copy(rows.at[1], shf.at[dstv.at[0]], ssem, add=True)

    def batch(t, u):
        q, p, qn, qf = u, u % 2, (u + 1) % 4, (u + 3) % 4
        _drain(table.at[pl.ds(0, EB)], rows.at[p], gsem)
        pltpu.async_copy(rows.at[p], shf.at[dstv.at[q]], ssem, add=True)
        _drain(table.at[pl.ds(0, EB)], rows.at[1 - p], ssem)
        _drain(srcf.at[pl.ds(base, EB)], srcv.at[qn], isem)
        _drain(dstf.at[pl.ds(base, EB)], dstv.at[qn], isem)
        pltpu.async_copy(table.at[srcv.at[qn]], rows.at[1 - p], gsem)
        off = base + (4 * t + u + 3) * EB
        pltpu.async_copy(srcf.at[pl.ds(off, EB)], srcv.at[qf], isem)
        pltpu.async_copy(dstf.at[pl.ds(off, EB)], dstv.at[qf], isem)

    def quad(t, carry):
        for u in range(4):
            batch(t, u)
        return carry

    lax.fori_loop(0, NB // 4, quad, 0)
    _drain(table.at[pl.ds(0, EB)], rows.at[0], gsem)
    _drain(table.at[pl.ds(0, EB)], rows.at[1], ssem)
    for q in range(2):
        _drain(srcf.at[pl.ds(base, EB)], srcv.at[q], isem)
        _drain(dstf.at[pl.ds(base, EB)], dstv.at[q], isem)


def _fire_dst(base, g, half, dstf, dstv, isem):
    for b in range(K):
        off = base + (g * K + b) * EB
        pltpu.async_copy(dstf.at[pl.ds(off, EB)], dstv.at[half * K + b], isem)


def _count_loop(base, dstf, shf, zf_hbm, ones_v, dstv, isem, ssem):
    _fire_dst(base, 0, 0, dstf, dstv, isem)

    def one_group(j, half):
        for b in range(K):
            _drain(dstf.at[pl.ds(base, EB)], dstv.at[half * K + b], isem)
        _fire_dst(base, j + 1, 1 - half, dstf, dstv, isem)
        for b in range(K):
            pltpu.async_copy(ones_v, shf.at[dstv.at[half * K + b]],
                             ssem, add=True)
        for b in range(K):
            _drain(zf_hbm.at[pl.ds(0, EB)], ones_v, ssem)

    def pair(t, carry):
        one_group(2 * t, 0)
        one_group(2 * t + 1, 1)
        return carry

    lax.fori_loop(0, NG // 2, pair, 0)
    for b in range(K):
        _drain(dstf.at[pl.ds(base, EB)], dstv.at[b], isem)


def _sc_seg_cnt(srcf_hbm, dstf_hbm, table_hbm, zf_hbm, ones_hbm,
                agg_out, cnt_out,
                shf, srcv, dstv, rows, isem, gsem, ssem):
    c = lax.axis_index("c")
    s = lax.axis_index("s")
    pltpu.sync_copy(zf_hbm.at[pl.ds(s * 632, 632)], shf.at[pl.ds(s * 632, 632)])
    plsc.subcore_barrier()
    base = c * EPAD + s * 20480
    _gather_scatter_loop(base, table_hbm, srcf_hbm, dstf_hbm, shf,
                         srcv, dstv, rows, zf_hbm, isem, gsem, ssem)
    plsc.subcore_barrier()
    _wb(s, shf, agg_out, c)
    plsc.subcore_barrier()
    pltpu.sync_copy(zf_hbm.at[pl.ds(s * 632, 632)], shf.at[pl.ds(s * 632, 632)])
    pltpu.sync_copy(ones_hbm, rows.at[0])
    plsc.subcore_barrier()
    _count_loop(base, dstf_hbm, shf, zf_hbm, rows.at[0], dstv, isem, ssem)
    plsc.subcore_barrier()
    _wb(s, shf, cnt_out, c)


def _sc_seg(srcf_hbm, dstf_hbm, table_hbm, zf_hbm,
            agg_out,
            shf, srcv, dstv, rows, isem, gsem, ssem):
    c = lax.axis_index("c")
    s = lax.axis_index("s")
    pltpu.sync_copy(zf_hbm.at[pl.ds(s * 632, 632)], shf.at[pl.ds(s * 632, 632)])
    plsc.subcore_barrier()
    base = c * EPAD + s * 20480
    _gather_scatter_loop(base, table_hbm, srcf_hbm, dstf_hbm, shf,
                         srcv, dstv, rows, zf_hbm, isem, gsem, ssem)
    plsc.subcore_barrier()
    _wb(s, shf, agg_out, c)


_SC_SCRATCH = (
    pltpu.VMEM_SHARED((NPAD, D), jnp.float32),
    pltpu.VMEM((2 * K, EB), jnp.int32),
    pltpu.VMEM((2 * K, EB), jnp.int32),
    pltpu.VMEM((K, EB, D), jnp.float32),
)
_SEMS = (pltpu.SemaphoreType.DMA,) * 3


def _seg_cnt(srcf, dstf, table, zf, ones):
    return pl.kernel(
        _sc_seg_cnt,
        out_type=(
            jax.ShapeDtypeStruct((2, N, D), jnp.float32),
            jax.ShapeDtypeStruct((2, N, D), jnp.float32),
        ),
        mesh=_mesh,
        scratch_types=_SC_SCRATCH + _SEMS,
    )(srcf, dstf, table, zf, ones)


def _seg(srcf, dstf, table, zf):
    return pl.kernel(
        _sc_seg,
        out_type=jax.ShapeDtypeStruct((2, N, D), jnp.float32),
        mesh=_mesh,
        scratch_types=_SC_SCRATCH + _SEMS,
    )(srcf, dstf, table, zf)


BN = 1000


def _tc1_body(p_ref, cp_ref, x_ref, wl_ref, wr_ref, b_ref, out_ref):
    cnt = cp_ref[0, :, 0:1]
    inv = 1.0 / jnp.maximum(cnt, 1.0)
    mean = p_ref[0] * inv
    r = (jnp.dot(mean, wl_ref[...], preferred_element_type=jnp.float32)
         + jnp.dot(x_ref[...], wr_ref[...], preferred_element_type=jnp.float32)
         + b_ref[...])
    h = jnp.maximum(r, 0.0)
    out_ref[0] = h[:, :D]
    out_ref[1] = h[:, D:]


def _tc1(rel, p, cp, x, wlT, wrT, b):
    return pl.pallas_call(
        _tc1_body,
        grid=(N // BN,),
        in_specs=[
            pl.BlockSpec((1, BN, D), lambda i, r=rel: (r, i, 0)),
            pl.BlockSpec((1, BN, D), lambda i, r=rel: (r, i, 0)),
            pl.BlockSpec((BN, D), lambda i: (i, 0)),
            pl.BlockSpec((D, H), lambda i: (0, 0)),
            pl.BlockSpec((D, H), lambda i: (0, 0)),
            pl.BlockSpec((1, H), lambda i: (0, 0)),
        ],
        out_specs=pl.BlockSpec((2, BN, D), lambda i: (0, i, 0)),
        out_shape=jax.ShapeDtypeStruct((2, N, D), jnp.float32),
    )(p, cp, x, wlT, wrT, b)


def _tc2_body(a_ref, cp_ref, h_ref, wl_ref, wr_ref, b_ref, out_ref):
    cnt = cp_ref[0, :, 0:1]
    inv = 1.0 / jnp.maximum(cnt, 1.0)
    r = (jnp.dot(a_ref[0] * inv, wl_ref[:D], preferred_element_type=jnp.float32)
         + jnp.dot(a_ref[1] * inv, wl_ref[D:], preferred_element_type=jnp.float32)
         + jnp.dot(h_ref[0], wr_ref[:D], preferred_element_type=jnp.float32)
         + jnp.dot(h_ref[1], wr_ref[D:], preferred_element_type=jnp.float32)
         + b_ref[...])
    out_ref[...] = jnp.maximum(r, 0.0)


def _tc2(rel, a, cp, h, wlT, wrT, b):
    return pl.pallas_call(
        _tc2_body,
        grid=(N // BN,),
        in_specs=[
            pl.BlockSpec((2, BN, D), lambda i: (0, i, 0)),
            pl.BlockSpec((1, BN, D), lambda i, r=rel: (r, i, 0)),
            pl.BlockSpec((2, BN, D), lambda i: (0, i, 0)),
            pl.BlockSpec((H, H), lambda i: (0, 0)),
            pl.BlockSpec((H, H), lambda i: (0, 0)),
            pl.BlockSpec((1, H), lambda i: (0, 0)),
        ],
        out_specs=pl.BlockSpec((BN, H), lambda i: (i, 0)),
        out_shape=jax.ShapeDtypeStruct((N, H), jnp.float32),
    )(a, cp, h, wlT, wrT, b)


def _tc2fin_body(a_ref, cp_ref, h_ref, wl_ref, wr_ref, b_ref, wlin_ref,
                 blin_ref, out_ref, log_ref):
    cnt = cp_ref[0, :, 0:1]
    inv = 1.0 / jnp.maximum(cnt, 1.0)
    r = (jnp.dot(a_ref[0] * inv, wl_ref[:D], preferred_element_type=jnp.float32)
         + jnp.dot(a_ref[1] * inv, wl_ref[D:], preferred_element_type=jnp.float32)
         + jnp.dot(h_ref[0], wr_ref[:D], preferred_element_type=jnp.float32)
         + jnp.dot(h_ref[1], wr_ref[D:], preferred_element_type=jnp.float32)
         + b_ref[...])
    c2 = jnp.maximum(r, 0.0)
    out_ref[...] = c2
    log_ref[...] = (jnp.dot(c2, wlin_ref[...], preferred_element_type=jnp.float32)
                    + blin_ref[...])


def _tc2fin(rel, a, cp, h, wlT, wrT, b, wlinT, blin):
    return pl.pallas_call(
        _tc2fin_body,
        grid=(N // BN,),
        in_specs=[
            pl.BlockSpec((2, BN, D), lambda i: (0, i, 0)),
            pl.BlockSpec((1, BN, D), lambda i, r=rel: (r, i, 0)),
            pl.BlockSpec((2, BN, D), lambda i: (0, i, 0)),
            pl.BlockSpec((H, H), lambda i: (0, 0)),
            pl.BlockSpec((H, H), lambda i: (0, 0)),
            pl.BlockSpec((1, H), lambda i: (0, 0)),
            pl.BlockSpec((H, OUT), lambda i: (0, 0)),
            pl.BlockSpec((1, OUT), lambda i: (0, 0)),
        ],
        out_specs=(
            pl.BlockSpec((BN, H), lambda i: (i, 0)),
            pl.BlockSpec((BN, OUT), lambda i: (i, 0)),
        ),
        out_shape=(
            jax.ShapeDtypeStruct((N, H), jnp.float32),
            jax.ShapeDtypeStruct((N, OUT), jnp.float32),
        ),
    )(a, cp, h, wlT, wrT, b, wlinT, blin)


def kernel(x_cell, x_gene, edge_index_ce, edge_index_ec,
           Wl1_cg, bl1_cg, Wr1_cg, Wl1_gc, bl1_gc, Wr1_gc,
           Wl2_cg, bl2_cg, Wr2_cg, Wl2_gc, bl2_gc, Wr2_gc,
           W_lin, b_lin):
    npad = EPAD - E
    tail = 4 * EB
    pad_src = jnp.zeros((npad,), jnp.int32)
    pad_dst = jnp.full((npad,), N, jnp.int32)
    zpad = jnp.zeros((tail,), jnp.int32)

    src_ce = jnp.concatenate([edge_index_ce[0].astype(jnp.int32), pad_src])
    dst_ce = jnp.concatenate([edge_index_ce[1].astype(jnp.int32), pad_dst])
    src_ec = jnp.concatenate([edge_index_ec[0].astype(jnp.int32), pad_src])
    dst_ec = jnp.concatenate([edge_index_ec[1].astype(jnp.int32), pad_dst])

    zf = jnp.zeros((NPAD, D), jnp.float32)
    ones = jnp.ones((EB, D), jnp.float32)

    x_both = jnp.concatenate([x_cell, x_gene], axis=0)
    src1 = jnp.concatenate([src_ce, src_ec + N, zpad])
    dst1 = jnp.concatenate([dst_ce, dst_ec, zpad])
    agg1, cnt1 = _seg_cnt(src1, dst1, x_both, zf, ones)

    g1 = _tc1(0, agg1, cnt1, x_gene, Wl1_cg.T, Wr1_cg.T, bl1_cg.reshape(1, H))
    c1 = _tc1(1, agg1, cnt1, x_cell, Wl1_gc.T, Wr1_gc.T, bl1_gc.reshape(1, H))

    src2_ce = jnp.concatenate([src_ce, src_ce + N, zpad])
    dst2_ce = jnp.concatenate([dst_ce, dst_ce, zpad])
    src2_ec = jnp.concatenate([src_ec, src_ec + N, zpad])
    dst2_ec = jnp.concatenate([dst_ec, dst_ec, zpad])
    a2g = _seg(src2_ce, dst2_ce, c1.reshape(2 * N, D), zf)
    a2c = _seg(src2_ec, dst2_ec, g1.reshape(2 * N, D), zf)

    g2 = _tc2(0, a2g, cnt1, g1, Wl2_cg.T, Wr2_cg.T, bl2_cg.reshape(1, H))
    c2, logits = _tc2fin(1, a2c, cnt1, c1, Wl2_gc.T, Wr2_gc.T,
                         bl2_gc.reshape(1, H), W_lin.T, b_lin.reshape(1, OUT))
    return (logits, c2, g2)
